# baseline (speedup 1.0000x reference)
import functools

import jax
import jax.numpy as jnp
from jax import lax
from jax.experimental import pallas as pl
from jax.experimental.pallas import tpu as pltpu
from jax.experimental.pallas import tpu_sc as plsc

N = 10000
NP = 10112
RPT = 632
E = 320000
T = 32
K = 128
B = 80
CH = 8
NC = K // CH
EP = T * K * B
D = 128
G = 64
C = 10
BN_EPS = 1e-5
BM = 1000
NBLK = N // BM


def _sc_degree_body(dst_hbm, zeros_hbm, ones_hbm, out_hbm, idx_v, ones_v, acc, _sem):
    c = lax.axis_index("c")
    s = lax.axis_index("s")
    wid = c * 16 + s
    lo = s * RPT
    pltpu.sync_copy(zeros_hbm, acc.at[pl.ds(lo, RPT)])
    pltpu.sync_copy(ones_hbm, ones_v)
    pltpu.sync_copy(dst_hbm.at[wid], idx_v)
    plsc.subcore_barrier()

    def step(j, carry):
        pltpu.sync_copy(ones_v, acc.at[idx_v.at[j]], add=True)
        return carry

    lax.fori_loop(0, K, step, 0)
    plsc.subcore_barrier()
    pltpu.sync_copy(acc.at[pl.ds(lo, RPT)], out_hbm.at[c, pl.ds(lo, RPT)])


@functools.cache
def _get_sc_degree():
    return functools.partial(
        pl.kernel,
        out_type=jax.ShapeDtypeStruct((2, NP, D), jnp.float32),
        scratch_types=[
            pltpu.VMEM((K, B), jnp.int32),
            pltpu.VMEM((B, D), jnp.float32),
            pltpu.VMEM_SHARED((NP, D), jnp.float32),
            pltpu.SemaphoreType.DMA,
        ],
        mesh=plsc.VectorSubcoreMesh(core_axis_name="c", subcore_axis_name="s"),
    )(_sc_degree_body)


def _sc_degree(*args):
    return _get_sc_degree()(*args)


def _sc_prop_body(h_hbm, src_hbm, dst_hbm, zeros_hbm, out_hbm,
                  idx_s, idx_d, buf0, buf1, acc, gsem, ssem, isem):
    c = lax.axis_index("c")
    s = lax.axis_index("s")
    wid = c * 16 + s
    lo = s * RPT
    pltpu.sync_copy(zeros_hbm, acc.at[pl.ds(lo, RPT)])
    plsc.subcore_barrier()

    gbuf = (buf0, buf1)

    def step(i, carry):
        i1 = pltpu.async_copy(src_hbm.at[wid, pl.ds(i * CH, CH)], idx_s, isem)
        i2 = pltpu.async_copy(dst_hbm.at[wid, pl.ds(i * CH, CH)], idx_d, isem)
        i1.wait()
        i2.wait()
        scats = []
        for b in range(CH):
            gb = gbuf[b % 2]
            pltpu.async_copy(h_hbm.at[idx_s.at[b]], gb, gsem).wait()
            scats.append(pltpu.async_copy(gb, acc.at[idx_d.at[b]], ssem, add=True))
            if b >= 1:
                scats[b - 1].wait()
        scats[CH - 1].wait()
        return carry

    lax.fori_loop(0, NC, step, 0)

    plsc.subcore_barrier()
    pltpu.sync_copy(acc.at[pl.ds(lo, RPT)], out_hbm.at[c, pl.ds(lo, RPT)])


@functools.cache
def _get_sc_prop():
    return functools.partial(
        pl.kernel,
        out_type=jax.ShapeDtypeStruct((2, NP, D), jnp.float32),
        scratch_types=[
            pltpu.VMEM((CH, B), jnp.int32),
            pltpu.VMEM((CH, B), jnp.int32),
            pltpu.VMEM((B, D), jnp.float32),
            pltpu.VMEM((B, D), jnp.float32),
            pltpu.VMEM_SHARED((NP, D), jnp.float32),
            pltpu.SemaphoreType.DMA,
            pltpu.SemaphoreType.DMA,
            pltpu.SemaphoreType.DMA,
        ],
        mesh=plsc.VectorSubcoreMesh(core_axis_name="c", subcore_axis_name="s"),
    )(_sc_prop_body)


def _sc_prop(*args):
    return _get_sc_prop()(*args)


def _dinv_of(d0_ref, d1_ref):
    deg = 1.0 + d0_ref[0][:, 0:1] + d1_ref[0][:, 0:1]
    return lax.rsqrt(jnp.maximum(deg, 1.0))


def _mm0_body(d0_ref, d1_ref, x_ref, w_ref, o_ref):
    dinv = _dinv_of(d0_ref, d1_ref)
    h = jnp.dot(x_ref[...], w_ref[...], preferred_element_type=jnp.float32)
    o_ref[...] = h * dinv


def _layer_body(d0_ref, d1_ref, p0_ref, p1_ref, ht_ref, w_ref, ab_ref, o_ref):
    dinv = _dinv_of(d0_ref, d1_ref)
    agg = dinv * (p0_ref[0] + p1_ref[0] + ht_ref[...])
    y = jnp.maximum(ab_ref[0:1, :] * agg + ab_ref[1:2, :], 0.0)
    o_ref[...] = jnp.dot(y, w_ref[...], preferred_element_type=jnp.float32) * dinv


def _head_body(d0_ref, d1_ref, p0_ref, p1_ref, ht_ref, b_ref, ab_ref,
               fc1_ref, fc1b_ref, fc2_ref, fc2b_ref, o_ref, accs, accc):
    i = pl.program_id(0)

    @pl.when(i == 0)
    def _():
        accs[...] = jnp.zeros_like(accs)
        accc[...] = jnp.zeros_like(accc)

    dinv = _dinv_of(d0_ref, d1_ref)
    agg = dinv * (p0_ref[0] + p1_ref[0] + ht_ref[...])
    y = jnp.maximum(ab_ref[0:1, :] * agg + ab_ref[1:2, :], 0.0)
    m = (b_ref[...] == lax.broadcasted_iota(jnp.int32, (BM, G), 1))
    m = m.astype(jnp.float32)
    accs[...] += lax.dot_general(m, y, (((0,), (0,)), ((), ())),
                                 preferred_element_type=jnp.float32)
    accc[...] += jnp.sum(m, axis=0)[:, None]

    @pl.when(i == NBLK - 1)
    def _():
        pooled = accs[...] / jnp.maximum(accc[...], 1.0)
        z = jnp.maximum(jnp.dot(pooled, fc1_ref[...],
                                preferred_element_type=jnp.float32) + fc1b_ref[...], 0.0)
        o_ref[...] = jnp.dot(z, fc2_ref[...],
                             preferred_element_type=jnp.float32) + fc2b_ref[...]


def _deg_specs():
    return [
        pl.BlockSpec((1, BM, 16), lambda i: (0, i, 0)),
        pl.BlockSpec((1, BM, 16), lambda i: (1, i, 0)),
    ]


def _mm0(pdeg, x, w):
    return pl.pallas_call(
        _mm0_body,
        grid=(NBLK,),
        in_specs=_deg_specs() + [
            pl.BlockSpec((BM, D), lambda i: (i, 0)),
            pl.BlockSpec((D, D), lambda i: (0, 0)),
        ],
        out_specs=pl.BlockSpec((BM, D), lambda i: (i, 0)),
        out_shape=jax.ShapeDtypeStruct((N, D), jnp.float32),
    )(pdeg, pdeg, x, w)


def _layer(pdeg, p, ht, w, ab):
    return pl.pallas_call(
        _layer_body,
        grid=(NBLK,),
        in_specs=_deg_specs() + [
            pl.BlockSpec((1, BM, D), lambda i: (0, i, 0)),
            pl.BlockSpec((1, BM, D), lambda i: (1, i, 0)),
            pl.BlockSpec((BM, D), lambda i: (i, 0)),
            pl.BlockSpec((D, D), lambda i: (0, 0)),
            pl.BlockSpec((2, D), lambda i: (0, 0)),
        ],
        out_specs=pl.BlockSpec((BM, D), lambda i: (i, 0)),
        out_shape=jax.ShapeDtypeStruct((N, D), jnp.float32),
    )(pdeg, pdeg, p, p, ht, w, ab)


def _head(pdeg, p, ht, bcol, ab, fc1_w, fc1_b, fc2_w, fc2_b):
    return pl.pallas_call(
        _head_body,
        grid=(NBLK,),
        in_specs=_deg_specs() + [
            pl.BlockSpec((1, BM, D), lambda i: (0, i, 0)),
            pl.BlockSpec((1, BM, D), lambda i: (1, i, 0)),
            pl.BlockSpec((BM, D), lambda i: (i, 0)),
            pl.BlockSpec((BM, 1), lambda i: (i, 0)),
            pl.BlockSpec((2, D), lambda i: (0, 0)),
            pl.BlockSpec((D, G), lambda i: (0, 0)),
            pl.BlockSpec((1, G), lambda i: (0, 0)),
            pl.BlockSpec((G, C), lambda i: (0, 0)),
            pl.BlockSpec((1, C), lambda i: (0, 0)),
        ],
        out_specs=pl.BlockSpec((G, C), lambda i: (0, 0)),
        out_shape=jax.ShapeDtypeStruct((G, C), jnp.float32),
        scratch_shapes=[
            pltpu.VMEM((G, D), jnp.float32),
            pltpu.VMEM((G, D), jnp.float32),
        ],
    )(pdeg, pdeg, p, p, ht, bcol, ab, fc1_w, fc1_b, fc2_w, fc2_b)


def kernel(x, edge_index, batch, W0, b0, g0, be0, W1, b1, g1, be1,
           W2, b2, g2, be2, fc1_w, fc1_b, fc2_w, fc2_b):
    pad = EP - E
    src = jnp.concatenate(
        [edge_index[0].astype(jnp.int32), jnp.zeros((pad,), jnp.int32)]
    ).reshape(T, K, B)
    dst = jnp.concatenate(
        [edge_index[1].astype(jnp.int32), jnp.full((pad,), N, jnp.int32)]
    ).reshape(T, K, B)
    bcol = batch.astype(jnp.int32).reshape(N, 1)

    onesD = jnp.ones((B, D), jnp.float32)
    zD = jnp.zeros((RPT, D), jnp.float32)

    inv_bn = 1.0 / jnp.sqrt(1.0 + BN_EPS)
    abs_ = []
    for (g, b, be) in ((g0, b0, be0), (g1, b1, be1), (g2, b2, be2)):
        alpha = g * inv_bn
        abs_.append(jnp.stack([alpha, alpha * b + be]))

    pdeg = _sc_degree(dst, zD, onesD)[:, :, :16]

    ht = _mm0(pdeg, x, W0)
    p = _sc_prop(ht, src, dst, zD)
    ht1 = _layer(pdeg, p, ht, W1, abs_[0])
    p = _sc_prop(ht1, src, dst, zD)
    ht2 = _layer(pdeg, p, ht1, W2, abs_[1])
    p = _sc_prop(ht2, src, dst, zD)
    return _head(pdeg, p, ht2, bcol, abs_[2], fc1_w, fc1_b.reshape(1, G),
                 fc2_w, fc2_b.reshape(1, C))


# speedup vs baseline: 2.5816x; 2.5816x over previous
import functools

import jax
import jax.numpy as jnp
from jax import lax
from jax.experimental import pallas as pl
from jax.experimental.pallas import tpu as pltpu
from jax.experimental.pallas import tpu_sc as plsc

N = 10000
NP = 10112
RPT = 632
E = 320000
T = 32
K = 80
B = 125
EP = T * K * B
D = 128
G = 64
C = 10
BN_EPS = 1e-5
BM = 1000
NBLK = N // BM


def _sc_degree_body(dst_hbm, zeros_hbm, ones_hbm, out_hbm, idx_v, ones_v, acc, _sem):
    c = lax.axis_index("c")
    s = lax.axis_index("s")
    wid = c * 16 + s
    lo = s * RPT
    pltpu.sync_copy(zeros_hbm, acc.at[pl.ds(lo, RPT)])
    pltpu.sync_copy(ones_hbm, ones_v)
    pltpu.sync_copy(dst_hbm.at[wid], idx_v)
    plsc.subcore_barrier()

    def step(j, carry):
        pltpu.sync_copy(ones_v, acc.at[idx_v.at[j]], add=True)
        return carry

    lax.fori_loop(0, K, step, 0)
    plsc.subcore_barrier()
    pltpu.sync_copy(acc.at[pl.ds(lo, RPT)], out_hbm.at[c, pl.ds(lo, RPT)])


@functools.cache
def _get_sc_degree():
    return functools.partial(
        pl.kernel,
        out_type=jax.ShapeDtypeStruct((2, NP, D), jnp.float32),
        scratch_types=[
            pltpu.VMEM((K, B), jnp.int32),
            pltpu.VMEM((B, D), jnp.float32),
            pltpu.VMEM_SHARED((NP, D), jnp.float32),
            pltpu.SemaphoreType.DMA,
        ],
        mesh=plsc.VectorSubcoreMesh(core_axis_name="c", subcore_axis_name="s"),
    )(_sc_degree_body)


def _sc_degree(*args):
    return _get_sc_degree()(*args)


def _sc_prop_body(h_hbm, src_hbm, dst_hbm, zeros_hbm, out_hbm,
                  idx_s, idx_d, buf0, acc, gsem):
    c = lax.axis_index("c")
    s = lax.axis_index("s")
    wid = c * 16 + s
    lo = s * RPT
    pltpu.sync_copy(zeros_hbm, acc.at[pl.ds(lo, RPT)])
    pltpu.sync_copy(src_hbm.at[wid], idx_s)
    pltpu.sync_copy(dst_hbm.at[wid], idx_d)
    plsc.subcore_barrier()

    def step(j, carry):
        pltpu.async_copy(h_hbm.at[idx_s.at[j]], buf0, gsem).wait()
        pltpu.sync_copy(buf0, acc.at[idx_d.at[j]], add=True)
        return carry

    lax.fori_loop(0, K, step, 0)

    plsc.subcore_barrier()
    pltpu.sync_copy(acc.at[pl.ds(lo, RPT)], out_hbm.at[c, pl.ds(lo, RPT)])


@functools.cache
def _get_sc_prop():
    return functools.partial(
        pl.kernel,
        out_type=jax.ShapeDtypeStruct((2, NP, D), jnp.float32),
        scratch_types=[
            pltpu.VMEM((K, B), jnp.int32),
            pltpu.VMEM((K, B), jnp.int32),
            pltpu.VMEM((B, D), jnp.float32),
            pltpu.VMEM_SHARED((NP, D), jnp.float32),
            pltpu.SemaphoreType.DMA,
        ],
        mesh=plsc.VectorSubcoreMesh(core_axis_name="c", subcore_axis_name="s"),
    )(_sc_prop_body)


def _sc_prop(*args):
    return _get_sc_prop()(*args)


def _dinv_of(d0_ref, d1_ref):
    deg = 1.0 + d0_ref[0][:, 0:1] + d1_ref[0][:, 0:1]
    return lax.rsqrt(jnp.maximum(deg, 1.0))


def _mm0_body(d0_ref, d1_ref, x_ref, w_ref, o_ref):
    dinv = _dinv_of(d0_ref, d1_ref)
    h = jnp.dot(x_ref[...], w_ref[...], preferred_element_type=jnp.float32)
    o_ref[...] = h * dinv


def _layer_body(d0_ref, d1_ref, p0_ref, p1_ref, ht_ref, w_ref, ab_ref, o_ref):
    dinv = _dinv_of(d0_ref, d1_ref)
    agg = dinv * (p0_ref[0] + p1_ref[0] + ht_ref[...])
    y = jnp.maximum(ab_ref[0:1, :] * agg + ab_ref[1:2, :], 0.0)
    o_ref[...] = jnp.dot(y, w_ref[...], preferred_element_type=jnp.float32) * dinv


def _head_body(d0_ref, d1_ref, p0_ref, p1_ref, ht_ref, b_ref, ab_ref,
               fc1_ref, fc1b_ref, fc2_ref, fc2b_ref, o_ref, accs, accc):
    i = pl.program_id(0)

    @pl.when(i == 0)
    def _():
        accs[...] = jnp.zeros_like(accs)
        accc[...] = jnp.zeros_like(accc)

    dinv = _dinv_of(d0_ref, d1_ref)
    agg = dinv * (p0_ref[0] + p1_ref[0] + ht_ref[...])
    y = jnp.maximum(ab_ref[0:1, :] * agg + ab_ref[1:2, :], 0.0)
    m = (b_ref[...] == lax.broadcasted_iota(jnp.int32, (BM, G), 1))
    m = m.astype(jnp.float32)
    accs[...] += lax.dot_general(m, y, (((0,), (0,)), ((), ())),
                                 preferred_element_type=jnp.float32)
    accc[...] += jnp.sum(m, axis=0)[:, None]

    @pl.when(i == NBLK - 1)
    def _():
        pooled = accs[...] / jnp.maximum(accc[...], 1.0)
        z = jnp.maximum(jnp.dot(pooled, fc1_ref[...],
                                preferred_element_type=jnp.float32) + fc1b_ref[...], 0.0)
        o_ref[...] = jnp.dot(z, fc2_ref[...],
                             preferred_element_type=jnp.float32) + fc2b_ref[...]


def _deg_specs():
    return [
        pl.BlockSpec((1, BM, 16), lambda i: (0, i, 0)),
        pl.BlockSpec((1, BM, 16), lambda i: (1, i, 0)),
    ]


def _mm0(pdeg, x, w):
    return pl.pallas_call(
        _mm0_body,
        grid=(NBLK,),
        in_specs=_deg_specs() + [
            pl.BlockSpec((BM, D), lambda i: (i, 0)),
            pl.BlockSpec((D, D), lambda i: (0, 0)),
        ],
        out_specs=pl.BlockSpec((BM, D), lambda i: (i, 0)),
        out_shape=jax.ShapeDtypeStruct((N, D), jnp.float32),
    )(pdeg, pdeg, x, w)


def _layer(pdeg, p, ht, w, ab):
    return pl.pallas_call(
        _layer_body,
        grid=(NBLK,),
        in_specs=_deg_specs() + [
            pl.BlockSpec((1, BM, D), lambda i: (0, i, 0)),
            pl.BlockSpec((1, BM, D), lambda i: (1, i, 0)),
            pl.BlockSpec((BM, D), lambda i: (i, 0)),
            pl.BlockSpec((D, D), lambda i: (0, 0)),
            pl.BlockSpec((2, D), lambda i: (0, 0)),
        ],
        out_specs=pl.BlockSpec((BM, D), lambda i: (i, 0)),
        out_shape=jax.ShapeDtypeStruct((N, D), jnp.float32),
    )(pdeg, pdeg, p, p, ht, w, ab)


def _head(pdeg, p, ht, bcol, ab, fc1_w, fc1_b, fc2_w, fc2_b):
    return pl.pallas_call(
        _head_body,
        grid=(NBLK,),
        in_specs=_deg_specs() + [
            pl.BlockSpec((1, BM, D), lambda i: (0, i, 0)),
            pl.BlockSpec((1, BM, D), lambda i: (1, i, 0)),
            pl.BlockSpec((BM, D), lambda i: (i, 0)),
            pl.BlockSpec((BM, 1), lambda i: (i, 0)),
            pl.BlockSpec((2, D), lambda i: (0, 0)),
            pl.BlockSpec((D, G), lambda i: (0, 0)),
            pl.BlockSpec((1, G), lambda i: (0, 0)),
            pl.BlockSpec((G, C), lambda i: (0, 0)),
            pl.BlockSpec((1, C), lambda i: (0, 0)),
        ],
        out_specs=pl.BlockSpec((G, C), lambda i: (0, 0)),
        out_shape=jax.ShapeDtypeStruct((G, C), jnp.float32),
        scratch_shapes=[
            pltpu.VMEM((G, D), jnp.float32),
            pltpu.VMEM((G, D), jnp.float32),
        ],
    )(pdeg, pdeg, p, p, ht, bcol, ab, fc1_w, fc1_b, fc2_w, fc2_b)


def kernel(x, edge_index, batch, W0, b0, g0, be0, W1, b1, g1, be1,
           W2, b2, g2, be2, fc1_w, fc1_b, fc2_w, fc2_b):
    pad = EP - E
    src = jnp.concatenate(
        [edge_index[0].astype(jnp.int32), jnp.zeros((pad,), jnp.int32)]
    ).reshape(T, K, B)
    dst = jnp.concatenate(
        [edge_index[1].astype(jnp.int32), jnp.full((pad,), N, jnp.int32)]
    ).reshape(T, K, B)
    bcol = batch.astype(jnp.int32).reshape(N, 1)

    onesD = jnp.ones((B, D), jnp.float32)
    zD = jnp.zeros((RPT, D), jnp.float32)

    inv_bn = 1.0 / jnp.sqrt(1.0 + BN_EPS)
    abs_ = []
    for (g, b, be) in ((g0, b0, be0), (g1, b1, be1), (g2, b2, be2)):
        alpha = g * inv_bn
        abs_.append(jnp.stack([alpha, alpha * b + be]))

    pdeg = _sc_degree(dst, zD, onesD)[:, :, :16]

    ht = _mm0(pdeg, x, W0)
    p = _sc_prop(ht, src, dst, zD)
    ht1 = _layer(pdeg, p, ht, W1, abs_[0])
    p = _sc_prop(ht1, src, dst, zD)
    ht2 = _layer(pdeg, p, ht1, W2, abs_[1])
    p = _sc_prop(ht2, src, dst, zD)
    return _head(pdeg, p, ht2, bcol, abs_[2], fc1_w, fc1_b.reshape(1, G),
                 fc2_w, fc2_b.reshape(1, C))
